# baseline (speedup 1.0000x reference)
import functools

import jax
import jax.numpy as jnp
from jax import lax
from jax.experimental import pallas as pl
from jax.experimental.pallas import tpu as pltpu
from jax.experimental.pallas import tpu_sc as plsc

_VOCAB = 262144
_HIDDEN = 1536
_NUM_TOKENS = 4 * 2048

_NUM_CORES = 2
_NUM_SUBCORES = 16
_NW = _NUM_CORES * _NUM_SUBCORES
_B_PER_W = _NUM_TOKENS // _NW
_CHUNK = 16
_NCHUNK = _B_PER_W // _CHUNK
_NBUF = 4
_NGROUP = _NCHUNK // _NBUF


def _make_gather():
  mesh = plsc.VectorSubcoreMesh(core_axis_name="c", subcore_axis_name="s")

  @functools.partial(
      pl.kernel,
      mesh=mesh,
      out_type=jax.ShapeDtypeStruct((_NUM_TOKENS, _HIDDEN), jnp.float32),
      scratch_types=[
          pltpu.VMEM((_NCHUNK, _CHUNK), jnp.int32),
          pltpu.VMEM((_NBUF, _CHUNK, _HIDDEN), jnp.float32),
          pltpu.SemaphoreType.DMA,
          pltpu.SemaphoreType.DMA,
          pltpu.SemaphoreType.DMA,
      ],
  )
  def gather_kernel(idx_hbm, table_hbm, out_hbm, idx_v, rows_v, isem, gsem,
                    ssem):
    wid = lax.axis_index("s") * _NUM_CORES + lax.axis_index("c")
    base = wid * _B_PER_W
    pltpu.async_copy(idx_hbm.at[wid], idx_v, isem).wait()

    def start_gather(n, b):
      pltpu.async_copy(table_hbm.at[idx_v.at[n]], rows_v.at[b], gsem)

    def wait_chunk(sem, b):
      pltpu.make_async_copy(out_hbm.at[pl.ds(0, _CHUNK)], rows_v.at[b],
                            sem).wait()

    for b in range(_NBUF):
      start_gather(b, b)

    def group(g, _):
      for b in range(_NBUF):
        wait_chunk(gsem, b)
        pltpu.async_copy(
            rows_v.at[b],
            out_hbm.at[pl.ds(base + (g * _NBUF + b) * _CHUNK, _CHUNK)],
            ssem)

      @pl.when(g + 1 < _NGROUP)
      def _prefetch():
        for b in range(_NBUF):
          wait_chunk(ssem, b)
          start_gather((g + 1) * _NBUF + b, b)

      return ()

    lax.fori_loop(0, _NGROUP, group, (), unroll=False)
    for b in range(_NBUF):
      wait_chunk(ssem, b)

  return gather_kernel


_gather = _make_gather()


def kernel(token_ids, table):
  ids = token_ids.astype(jnp.int32).reshape(_NW, _NCHUNK, _CHUNK)
  out = _gather(ids, table)
  return out.reshape(token_ids.shape[0], token_ids.shape[1], _HIDDEN)
